# baseline (speedup 1.0000x reference)
import functools

import jax
import jax.numpy as jnp
from jax import lax
from jax.experimental import pallas as pl
from jax.experimental.pallas import tpu as pltpu
from jax.experimental.pallas import tpu_sc as plsc

N = 10000
E = 160000
D_IN = 32
D_EDGE = 16
HIDDEN = 16
NUM_GRAPHS = 64
NUM_CLASSES = 8

NC = 2
NS = 16
NW = NC * NS
CHUNK = 128
IDX_C = 20
GCH = 640
IDX_G = 4
E_PER_W = IDX_C * CHUNK
E_HALF = NW * E_PER_W
E_PAD = 2 * E_HALF
N_PAD = 10240
RPT = N_PAD // NS

_mesh_cache = []


def _mesh():
    if not _mesh_cache:
        _mesh_cache.append(plsc.VectorSubcoreMesh(
            core_axis_name="c", subcore_axis_name="s",
            num_cores=NC, num_subcores=NS))
    return _mesh_cache[0]

_f32 = jnp.float32


def _sc_gather_rows(tab, src3, d):
    @functools.partial(
        pl.kernel,
        out_type=jax.ShapeDtypeStruct((E_HALF, d), _f32),
        mesh=_mesh(),
        compiler_params=pltpu.CompilerParams(use_tc_tiling_on_sc=False),
        scratch_types=[
            pltpu.VMEM((IDX_G, GCH), jnp.int32),
            pltpu.VMEM((GCH, d), _f32),
            pltpu.SemaphoreType.DMA,
        ],
    )
    def body(tab_hbm, src_hbm, out_hbm, src_v, rows_v, sem):
        c = lax.axis_index("c")
        s = lax.axis_index("s")
        wid = s * NC + c
        base = pl.multiple_of(wid * E_PER_W, 8)
        pltpu.sync_copy(src_hbm.at[wid], src_v)

        def gbody(j, carry):
            off = pl.multiple_of(base + j * GCH, 8)
            pltpu.async_copy(tab_hbm.at[src_v.at[j]], rows_v, sem).wait()
            pltpu.sync_copy(rows_v, out_hbm.at[pl.ds(off, GCH)])
            return carry

        lax.fori_loop(0, IDX_G, gbody, 0)

    return body(tab, src3)


def _sc_scatter(msg, dst3, zeros_s):
    @functools.partial(
        pl.kernel,
        out_type=jax.ShapeDtypeStruct((NC, N_PAD, HIDDEN), _f32),
        mesh=_mesh(),
        compiler_params=pltpu.CompilerParams(use_tc_tiling_on_sc=False),
        scratch_types=[
            pltpu.VMEM((IDX_C, CHUNK), jnp.int32),
            pltpu.VMEM((CHUNK, HIDDEN), _f32),
            pltpu.VMEM_SHARED((N_PAD, HIDDEN), _f32),
        ],
    )
    def body(msg_hbm, dst_hbm, zeros_hbm, out_hbm, dst_v, msg_v, acc):
        c = lax.axis_index("c")
        s = lax.axis_index("s")
        wid = s * NC + c
        base = pl.multiple_of(wid * E_PER_W, 8)
        pltpu.sync_copy(dst_hbm.at[wid], dst_v)
        pltpu.sync_copy(zeros_hbm, acc.at[pl.ds(s * RPT, RPT)])
        plsc.subcore_barrier()

        def sbody(j, carry):
            off = pl.multiple_of(base + j * CHUNK, 8)
            pltpu.sync_copy(msg_hbm.at[pl.ds(off, CHUNK)], msg_v)
            pltpu.sync_copy(msg_v, acc.at[dst_v.at[j]], add=True)
            return carry

        lax.fori_loop(0, IDX_C, sbody, 0)
        plsc.subcore_barrier()
        pltpu.sync_copy(acc.at[pl.ds(s * RPT, RPT)],
                        out_hbm.at[c, pl.ds(s * RPT, RPT)])

    return body(msg, dst3, zeros_s)


def _sc_scatter_count(msg, dst3, ones_c, zeros_s):
    @functools.partial(
        pl.kernel,
        out_type=[
            jax.ShapeDtypeStruct((NC, N_PAD, HIDDEN), _f32),
            jax.ShapeDtypeStruct((NC, N_PAD, HIDDEN), _f32),
        ],
        mesh=_mesh(),
        compiler_params=pltpu.CompilerParams(use_tc_tiling_on_sc=False),
        scratch_types=[
            pltpu.VMEM((IDX_C, CHUNK), jnp.int32),
            pltpu.VMEM((CHUNK, HIDDEN), _f32),
            pltpu.VMEM((CHUNK, HIDDEN), _f32),
            pltpu.VMEM_SHARED((N_PAD, HIDDEN), _f32),
            pltpu.VMEM_SHARED((N_PAD, HIDDEN), _f32),
        ],
    )
    def body(msg_hbm, dst_hbm, ones_hbm, zeros_hbm, out_hbm, cnt_hbm,
             dst_v, msg_v, ones_v, acc, cacc):
        c = lax.axis_index("c")
        s = lax.axis_index("s")
        wid = s * NC + c
        base = pl.multiple_of(wid * E_PER_W, 8)
        pltpu.sync_copy(dst_hbm.at[wid], dst_v)
        pltpu.sync_copy(ones_hbm, ones_v)
        pltpu.sync_copy(zeros_hbm, acc.at[pl.ds(s * RPT, RPT)])
        pltpu.sync_copy(zeros_hbm, cacc.at[pl.ds(s * RPT, RPT)])
        plsc.subcore_barrier()

        def sbody(j, carry):
            off = pl.multiple_of(base + j * CHUNK, 8)
            pltpu.sync_copy(msg_hbm.at[pl.ds(off, CHUNK)], msg_v)
            pltpu.sync_copy(msg_v, acc.at[dst_v.at[j]], add=True)
            pltpu.sync_copy(ones_v, cacc.at[dst_v.at[j]], add=True)
            return carry

        lax.fori_loop(0, IDX_C, sbody, 0)
        plsc.subcore_barrier()
        pltpu.sync_copy(acc.at[pl.ds(s * RPT, RPT)],
                        out_hbm.at[c, pl.ds(s * RPT, RPT)])
        pltpu.sync_copy(cacc.at[pl.ds(s * RPT, RPT)],
                        cnt_hbm.at[c, pl.ds(s * RPT, RPT)])

    return body(msg, dst3, ones_c, zeros_s)


_BE = 2048
_B4 = _BE // 4
_B8 = _BE // 8
_EA_LAST = (E // 8) // _B8


_bf16 = jnp.bfloat16


def _edge0_body(ea_ref, xg_ref, pe_ref, po_ref, We_ref, be_ref, R_ref, S_ref,
                out_ref):
    w = jnp.maximum(
        jnp.dot(ea_ref[...], We_ref[...], preferred_element_type=_f32)
        + be_ref[...], 0.0).astype(_bf16)
    xgb = xg_ref[...].astype(_bf16)
    lo = jnp.dot(pe_ref[...], xgb, preferred_element_type=_f32)
    hi = jnp.dot(po_ref[...], xgb, preferred_element_type=_f32)
    xg8 = jnp.concatenate([lo, hi], axis=1).astype(_bf16)
    xr = jnp.dot(xg8, R_ref[...], preferred_element_type=_f32).astype(_bf16)
    out_ref[...] = jnp.dot(xr * w, S_ref[...], preferred_element_type=_f32)


def _edge1_body(ea_ref, hg_ref, We_ref, be_ref, R_ref, S_ref, out_ref):
    w = jnp.maximum(
        jnp.dot(ea_ref[...], We_ref[...], preferred_element_type=_f32)
        + be_ref[...], 0.0).astype(_bf16)
    xr = jnp.dot(hg_ref[...].astype(_bf16), R_ref[...],
                 preferred_element_type=_f32).astype(_bf16)
    out_ref[...] = jnp.dot(xr * w, S_ref[...], preferred_element_type=_f32)


def _bcast_spec(arr):
    return pl.BlockSpec(arr.shape, lambda i: tuple(0 for _ in arr.shape))


def _edge_conv0(ea8, xg4, pe, po, We8, be8, R8, S8, off):
    n_blocks = E_HALF // _BE
    return pl.pallas_call(
        _edge0_body,
        grid=(n_blocks,),
        in_specs=[
            pl.BlockSpec((_B8, 128),
                         lambda i: (jnp.minimum(i + off, _EA_LAST), 0)),
            pl.BlockSpec((_B4, 128), lambda i: (i, 0)),
            _bcast_spec(pe), _bcast_spec(po), _bcast_spec(We8),
            pl.BlockSpec((1, We8.shape[1]), lambda i: (0, 0)),
            _bcast_spec(R8), _bcast_spec(S8),
        ],
        out_specs=pl.BlockSpec((_B8, 128), lambda i: (i, 0)),
        out_shape=jax.ShapeDtypeStruct((E_HALF // 8, 128), _f32),
    )(ea8, xg4, pe, po, We8, be8.reshape(1, -1), R8, S8)


def _edge_conv1(ea8, hg8, We8, be8, R8, S8, off):
    n_blocks = E_HALF // _BE
    return pl.pallas_call(
        _edge1_body,
        grid=(n_blocks,),
        in_specs=[
            pl.BlockSpec((_B8, 128),
                         lambda i: (jnp.minimum(i + off, _EA_LAST), 0)),
            pl.BlockSpec((_B8, 128), lambda i: (i, 0)),
            _bcast_spec(We8),
            pl.BlockSpec((1, We8.shape[1]), lambda i: (0, 0)),
            _bcast_spec(R8), _bcast_spec(S8),
        ],
        out_specs=pl.BlockSpec((_B8, 128), lambda i: (i, 0)),
        out_shape=jax.ShapeDtypeStruct((E_HALF // 8, 128), _f32),
    )(ea8, hg8, We8, be8.reshape(1, -1), R8, S8)


def _upd_body(sa0, sa1, sb0, sb1, ca0, ca1, cb0, cb1,
              x_ref, root_ref, b_ref, out_ref):
    cnt = jnp.maximum(
        (ca0[...] + ca1[...] + cb0[...] + cb1[...])[:, :1], 1.0)
    agg = (sa0[...] + sa1[...] + sb0[...] + sb1[...]) / cnt
    out_ref[...] = agg + jnp.dot(
        x_ref[...], root_ref[...], preferred_element_type=_f32) + b_ref[...]


def _node_update(s_parts, c_parts, x, root, bias):
    return pl.pallas_call(
        _upd_body,
        out_shape=jax.ShapeDtypeStruct((N, HIDDEN), _f32),
    )(s_parts[0][0, :N], s_parts[0][1, :N],
      s_parts[1][0, :N], s_parts[1][1, :N],
      c_parts[0][0, :N], c_parts[0][1, :N],
      c_parts[1][0, :N], c_parts[1][1, :N],
      x, root, bias.reshape(1, HIDDEN))


def _final_body(sa0, sa1, sb0, sb1, ca0, ca1, cb0, cb1, h_ref, root_ref,
                b_ref, bt_ref, fc0W_ref, fc0b_ref, fc1W_ref, fc1b_ref,
                out_ref):
    cnt = jnp.maximum(
        (ca0[...] + ca1[...] + cb0[...] + cb1[...])[:, :1], 1.0)
    h2 = ((sa0[...] + sa1[...] + sb0[...] + sb1[...]) / cnt
          + jnp.dot(h_ref[...], root_ref[...], preferred_element_type=_f32)
          + b_ref[...])
    oh = (bt_ref[...] == lax.broadcasted_iota(
        jnp.int32, (NUM_GRAPHS, N), 0)).astype(_f32)
    gs = jnp.dot(oh, h2, preferred_element_type=_f32)
    gc = jnp.sum(oh, axis=1, keepdims=True)
    p = gs / jnp.maximum(gc, 1.0)
    p = jnp.maximum(
        jnp.dot(p, fc0W_ref[...], preferred_element_type=_f32)
        + fc0b_ref[...], 0.0)
    out_ref[...] = jnp.dot(
        p, fc1W_ref[...], preferred_element_type=_f32) + fc1b_ref[...]


def _final(s_parts, c_parts, h, root, bias, batch, fc0W, fc0b, fc1W, fc1b):
    return pl.pallas_call(
        _final_body,
        out_shape=jax.ShapeDtypeStruct((NUM_GRAPHS, NUM_CLASSES), _f32),
    )(s_parts[0][0, :N], s_parts[0][1, :N],
      s_parts[1][0, :N], s_parts[1][1, :N],
      c_parts[0][0, :N], c_parts[0][1, :N],
      c_parts[1][0, :N], c_parts[1][1, :N],
      h, root, bias.reshape(1, HIDDEN), batch.reshape(1, N),
      fc0W, fc0b.reshape(1, HIDDEN), fc1W, fc1b.reshape(1, NUM_CLASSES))


def kernel(x, edge_index, edge_attr, batch, We0, be0, root0, bias0,
           We1, be1, root1, bias1, fc0W, fc0b, fc1W, fc1b):
    src = edge_index[0]
    dst = edge_index[1]
    pad = E_PAD - E
    srcg = jnp.concatenate(
        [src, jnp.zeros((pad,), jnp.int32)]).reshape(2, NW, IDX_G, GCH)
    dst3 = jnp.concatenate(
        [dst, jnp.full((pad,), N, jnp.int32)]).reshape(2, NW, IDX_C, CHUNK)
    ones_c = jnp.ones((CHUNK, HIDDEN), _f32)
    zeros_s = jnp.zeros((RPT, HIDDEN), _f32)
    I8 = jnp.eye(8, dtype=_f32)
    R0 = jnp.kron(jnp.eye(D_IN, dtype=_f32), jnp.ones((1, HIDDEN), _f32))
    S0 = jnp.kron(jnp.ones((D_IN, 1), _f32), jnp.eye(HIDDEN, dtype=_f32))
    R1 = jnp.kron(jnp.eye(HIDDEN, dtype=_f32), jnp.ones((1, HIDDEN), _f32))
    S1 = jnp.kron(jnp.ones((HIDDEN, 1), _f32), jnp.eye(HIDDEN, dtype=_f32))
    We0_8 = jnp.kron(I8, We0).astype(_bf16)
    We1_8 = jnp.kron(I8, We1).astype(_bf16)
    R0_8 = jnp.kron(I8, R0).astype(_bf16)
    R1_8 = jnp.kron(I8, R1).astype(_bf16)
    S0_8 = jnp.kron(I8, S0).astype(_bf16)
    S1_8 = jnp.kron(I8, S1).astype(_bf16)
    be0_8 = jnp.tile(be0, 8)
    be1_8 = jnp.tile(be1, 8)
    ea8 = edge_attr.reshape(E // 8, 128).astype(_bf16)
    pe = jnp.kron(jnp.eye(_B8, dtype=_bf16), jnp.array([[1, 0]], _bf16))
    po = jnp.kron(jnp.eye(_B8, dtype=_bf16), jnp.array([[0, 1]], _bf16))

    halves = (0, 1)
    off_b = E_HALF // _BE
    xgs = [_sc_gather_rows(x, srcg[hh], D_IN) for hh in halves]
    msg0 = [_edge_conv0(ea8, xgs[hh].reshape(E_HALF // 4, 128), pe, po,
                        We0_8, be0_8, R0_8, S0_8, hh * off_b)
            for hh in halves]
    sc0 = [_sc_scatter_count(msg0[hh].reshape(E_HALF, HIDDEN), dst3[hh],
                             ones_c, zeros_s) for hh in halves]
    s0_parts = [sc0[0][0], sc0[1][0]]
    cnt_parts = [sc0[0][1], sc0[1][1]]
    h = _node_update(s0_parts, cnt_parts, x, root0, bias0)
    hgs = [_sc_gather_rows(h, srcg[hh], HIDDEN) for hh in halves]
    msg1 = [_edge_conv1(ea8, hgs[hh].reshape(E_HALF // 8, 128),
                        We1_8, be1_8, R1_8, S1_8, hh * off_b)
            for hh in halves]
    s1_parts = [_sc_scatter(msg1[hh].reshape(E_HALF, HIDDEN), dst3[hh],
                            zeros_s) for hh in halves]
    return _final(s1_parts, cnt_parts, h, root1, bias1, batch,
                  fc0W, fc0b, fc1W, fc1b)


# speedup vs baseline: 1.0599x; 1.0599x over previous
import functools

import jax
import jax.numpy as jnp
from jax import lax
from jax.experimental import pallas as pl
from jax.experimental.pallas import tpu as pltpu
from jax.experimental.pallas import tpu_sc as plsc

N = 10000
E = 160000
D_IN = 32
D_EDGE = 16
HIDDEN = 16
NUM_GRAPHS = 64
NUM_CLASSES = 8

NC = 2
NS = 16
NW = NC * NS
CHUNK = 128
IDX_C = 20
GCH = 640
IDX_G = 4
E_PER_W = IDX_C * CHUNK
E_HALF = NW * E_PER_W
E_PAD = 2 * E_HALF
N_PAD = 10240
RPT = N_PAD // NS

_mesh_cache = []


def _mesh():
    if not _mesh_cache:
        _mesh_cache.append(plsc.VectorSubcoreMesh(
            core_axis_name="c", subcore_axis_name="s",
            num_cores=NC, num_subcores=NS))
    return _mesh_cache[0]

_f32 = jnp.float32


def _sc_gather_rows(tab, src3, d):
    @functools.partial(
        pl.kernel,
        out_type=jax.ShapeDtypeStruct((E_HALF, d), _f32),
        mesh=_mesh(),
        compiler_params=pltpu.CompilerParams(use_tc_tiling_on_sc=False),
        scratch_types=[
            pltpu.VMEM((IDX_G, GCH), jnp.int32),
            pltpu.VMEM((GCH, d), _f32),
            pltpu.SemaphoreType.DMA,
        ],
    )
    def body(tab_hbm, src_hbm, out_hbm, src_v, rows_v, sem):
        c = lax.axis_index("c")
        s = lax.axis_index("s")
        wid = s * NC + c
        base = pl.multiple_of(wid * E_PER_W, 8)
        pltpu.sync_copy(src_hbm.at[wid], src_v)

        def gbody(j, carry):
            off = pl.multiple_of(base + j * GCH, 8)
            pltpu.async_copy(tab_hbm.at[src_v.at[j]], rows_v, sem).wait()
            pltpu.sync_copy(rows_v, out_hbm.at[pl.ds(off, GCH)])
            return carry

        lax.fori_loop(0, IDX_G, gbody, 0)

    return body(tab, src3)


def _sc_scatter(msg, dst3, zeros_s):
    @functools.partial(
        pl.kernel,
        out_type=jax.ShapeDtypeStruct((NC, N_PAD, HIDDEN), _f32),
        mesh=_mesh(),
        compiler_params=pltpu.CompilerParams(use_tc_tiling_on_sc=False),
        scratch_types=[
            pltpu.VMEM((IDX_C, CHUNK), jnp.int32),
            pltpu.VMEM((CHUNK, HIDDEN), _f32),
            pltpu.VMEM_SHARED((N_PAD, HIDDEN), _f32),
        ],
    )
    def body(msg_hbm, dst_hbm, zeros_hbm, out_hbm, dst_v, msg_v, acc):
        c = lax.axis_index("c")
        s = lax.axis_index("s")
        wid = s * NC + c
        base = pl.multiple_of(wid * E_PER_W, 8)
        pltpu.sync_copy(dst_hbm.at[wid], dst_v)
        pltpu.sync_copy(zeros_hbm, acc.at[pl.ds(s * RPT, RPT)])
        plsc.subcore_barrier()

        def sbody(j, carry):
            off = pl.multiple_of(base + j * CHUNK, 8)
            pltpu.sync_copy(msg_hbm.at[pl.ds(off, CHUNK)], msg_v)
            pltpu.sync_copy(msg_v, acc.at[dst_v.at[j]], add=True)
            return carry

        lax.fori_loop(0, IDX_C, sbody, 0)
        plsc.subcore_barrier()
        pltpu.sync_copy(acc.at[pl.ds(s * RPT, RPT)],
                        out_hbm.at[c, pl.ds(s * RPT, RPT)])

    return body(msg, dst3, zeros_s)


def _sc_scatter_count(msg, dst3, ones_c, zeros_s):
    @functools.partial(
        pl.kernel,
        out_type=[
            jax.ShapeDtypeStruct((NC, N_PAD, HIDDEN), _f32),
            jax.ShapeDtypeStruct((NC, N_PAD, HIDDEN), _f32),
        ],
        mesh=_mesh(),
        compiler_params=pltpu.CompilerParams(use_tc_tiling_on_sc=False),
        scratch_types=[
            pltpu.VMEM((IDX_C, CHUNK), jnp.int32),
            pltpu.VMEM((CHUNK, HIDDEN), _f32),
            pltpu.VMEM((CHUNK, HIDDEN), _f32),
            pltpu.VMEM_SHARED((N_PAD, HIDDEN), _f32),
            pltpu.VMEM_SHARED((N_PAD, HIDDEN), _f32),
        ],
    )
    def body(msg_hbm, dst_hbm, ones_hbm, zeros_hbm, out_hbm, cnt_hbm,
             dst_v, msg_v, ones_v, acc, cacc):
        c = lax.axis_index("c")
        s = lax.axis_index("s")
        wid = s * NC + c
        base = pl.multiple_of(wid * E_PER_W, 8)
        pltpu.sync_copy(dst_hbm.at[wid], dst_v)
        pltpu.sync_copy(ones_hbm, ones_v)
        pltpu.sync_copy(zeros_hbm, acc.at[pl.ds(s * RPT, RPT)])
        pltpu.sync_copy(zeros_hbm, cacc.at[pl.ds(s * RPT, RPT)])
        plsc.subcore_barrier()

        def sbody(j, carry):
            off = pl.multiple_of(base + j * CHUNK, 8)
            pltpu.sync_copy(msg_hbm.at[pl.ds(off, CHUNK)], msg_v)
            pltpu.sync_copy(msg_v, acc.at[dst_v.at[j]], add=True)
            pltpu.sync_copy(ones_v, cacc.at[dst_v.at[j]], add=True)
            return carry

        lax.fori_loop(0, IDX_C, sbody, 0)
        plsc.subcore_barrier()
        pltpu.sync_copy(acc.at[pl.ds(s * RPT, RPT)],
                        out_hbm.at[c, pl.ds(s * RPT, RPT)])
        pltpu.sync_copy(cacc.at[pl.ds(s * RPT, RPT)],
                        cnt_hbm.at[c, pl.ds(s * RPT, RPT)])

    return body(msg, dst3, ones_c, zeros_s)


_BE = 2048
_B4 = _BE // 4
_B8 = _BE // 8
_EA_LAST = (E // 8) // _B8


_bf16 = jnp.bfloat16


def _edge0_body(ea_ref, xg_ref, pe_ref, po_ref, We_ref, be_ref, R_ref, S_ref,
                out_ref):
    w = jnp.maximum(
        jnp.dot(ea_ref[...], We_ref[...], preferred_element_type=_f32)
        + be_ref[...], 0.0).astype(_bf16)
    xgb = xg_ref[...].astype(_bf16)
    lo = jnp.dot(pe_ref[...], xgb, preferred_element_type=_f32)
    hi = jnp.dot(po_ref[...], xgb, preferred_element_type=_f32)
    xg8 = jnp.concatenate([lo, hi], axis=1).astype(_bf16)
    xr = jnp.dot(xg8, R_ref[...], preferred_element_type=_f32).astype(_bf16)
    out_ref[...] = jnp.dot(xr * w, S_ref[...], preferred_element_type=_f32)


def _edge1_body(ea_ref, hg_ref, We_ref, be_ref, R_ref, S_ref, out_ref):
    w = jnp.maximum(
        jnp.dot(ea_ref[...], We_ref[...], preferred_element_type=_f32)
        + be_ref[...], 0.0).astype(_bf16)
    xr = jnp.dot(hg_ref[...].astype(_bf16), R_ref[...],
                 preferred_element_type=_f32).astype(_bf16)
    out_ref[...] = jnp.dot(xr * w, S_ref[...], preferred_element_type=_f32)


def _bcast_spec(arr):
    return pl.BlockSpec(arr.shape, lambda i: tuple(0 for _ in arr.shape))


def _edge_conv0(ea8, xg4, pe, po, We8, be8, R8, S8, off):
    n_blocks = E_HALF // _BE
    return pl.pallas_call(
        _edge0_body,
        grid=(n_blocks,),
        in_specs=[
            pl.BlockSpec((_B8, 128),
                         lambda i: (jnp.minimum(i + off, _EA_LAST), 0)),
            pl.BlockSpec((_B4, 128), lambda i: (i, 0)),
            _bcast_spec(pe), _bcast_spec(po), _bcast_spec(We8),
            pl.BlockSpec((1, We8.shape[1]), lambda i: (0, 0)),
            _bcast_spec(R8), _bcast_spec(S8),
        ],
        out_specs=pl.BlockSpec((_B8, 128), lambda i: (i, 0)),
        out_shape=jax.ShapeDtypeStruct((E_HALF // 8, 128), _f32),
    )(ea8, xg4, pe, po, We8, be8.reshape(1, -1), R8, S8)


def _edge_conv1(ea8, hg8, We8, be8, R8, S8, off):
    n_blocks = E_HALF // _BE
    return pl.pallas_call(
        _edge1_body,
        grid=(n_blocks,),
        in_specs=[
            pl.BlockSpec((_B8, 128),
                         lambda i: (jnp.minimum(i + off, _EA_LAST), 0)),
            pl.BlockSpec((_B8, 128), lambda i: (i, 0)),
            _bcast_spec(We8),
            pl.BlockSpec((1, We8.shape[1]), lambda i: (0, 0)),
            _bcast_spec(R8), _bcast_spec(S8),
        ],
        out_specs=pl.BlockSpec((_B8, 128), lambda i: (i, 0)),
        out_shape=jax.ShapeDtypeStruct((E_HALF // 8, 128), _f32),
    )(ea8, hg8, We8, be8.reshape(1, -1), R8, S8)


_NP8 = N_PAD // 8


def _upd_body(sa0, sa1, sb0, sb1, ca0, ca1, cb0, cb1,
              x8_ref, root8_ref, b_ref, out_ref):
    cnt = jnp.maximum(ca0[...] + ca1[...] + cb0[...] + cb1[...], 1.0)
    agg = (sa0[...] + sa1[...] + sb0[...] + sb1[...]) / cnt
    out_ref[...] = agg + jnp.dot(
        x8_ref[...], root8_ref[...], preferred_element_type=_f32) + b_ref[...]


def _node_update(s_parts, c_parts, x8, root8, bias8):
    sp = [p.reshape(NC, _NP8, 128) for p in s_parts]
    cp = [p.reshape(NC, _NP8, 128) for p in c_parts]
    return pl.pallas_call(
        _upd_body,
        out_shape=jax.ShapeDtypeStruct((_NP8, 128), _f32),
    )(sp[0][0], sp[0][1], sp[1][0], sp[1][1],
      cp[0][0], cp[0][1], cp[1][0], cp[1][1],
      x8, root8, bias8.reshape(1, 128))


def _final_body(sa0, sa1, sb0, sb1, ca0, ca1, cb0, cb1, h8_ref, root8_ref,
                b_ref, btT_ref, fc0W_ref, fc0b_ref, fc1W_ref, fc1b_ref,
                out_ref):
    cnt = jnp.maximum(ca0[...] + ca1[...] + cb0[...] + cb1[...], 1.0)
    h2 = ((sa0[...] + sa1[...] + sb0[...] + sb1[...]) / cnt
          + jnp.dot(h8_ref[...], root8_ref[...], preferred_element_type=_f32)
          + b_ref[...])
    rmask = lax.broadcasted_iota(jnp.int32, (_NP8, 128), 0) < (N // 8)
    h2 = jnp.where(rmask, h2, 0.0)
    gs = jnp.zeros((NUM_GRAPHS, HIDDEN), _f32)
    gc = jnp.zeros((NUM_GRAPHS, 1), _f32)
    giota = lax.broadcasted_iota(jnp.int32, (NUM_GRAPHS, _NP8), 0)
    for j in range(8):
        ohj = (btT_ref[j:j + 1, :] == giota).astype(_f32)
        gs = gs + jnp.dot(ohj, h2[:, j * HIDDEN:(j + 1) * HIDDEN],
                          preferred_element_type=_f32)
        gc = gc + jnp.sum(ohj, axis=1, keepdims=True)
    p = gs / jnp.maximum(gc, 1.0)
    p = jnp.maximum(
        jnp.dot(p, fc0W_ref[...], preferred_element_type=_f32)
        + fc0b_ref[...], 0.0)
    out_ref[...] = jnp.dot(
        p, fc1W_ref[...], preferred_element_type=_f32) + fc1b_ref[...]


def _final(s_parts, c_parts, h8, root8, bias8, btT,
           fc0W, fc0b, fc1W, fc1b):
    sp = [p.reshape(NC, _NP8, 128) for p in s_parts]
    cp = [p.reshape(NC, _NP8, 128) for p in c_parts]
    return pl.pallas_call(
        _final_body,
        out_shape=jax.ShapeDtypeStruct((NUM_GRAPHS, NUM_CLASSES), _f32),
    )(sp[0][0], sp[0][1], sp[1][0], sp[1][1],
      cp[0][0], cp[0][1], cp[1][0], cp[1][1],
      h8, root8, bias8.reshape(1, 128), btT,
      fc0W, fc0b.reshape(1, HIDDEN), fc1W, fc1b.reshape(1, NUM_CLASSES))


def kernel(x, edge_index, edge_attr, batch, We0, be0, root0, bias0,
           We1, be1, root1, bias1, fc0W, fc0b, fc1W, fc1b):
    src = edge_index[0]
    dst = edge_index[1]
    pad = E_PAD - E
    srcg = jnp.concatenate(
        [src, jnp.zeros((pad,), jnp.int32)]).reshape(2, NW, IDX_G, GCH)
    dst3 = jnp.concatenate(
        [dst, jnp.full((pad,), N, jnp.int32)]).reshape(2, NW, IDX_C, CHUNK)
    ones_c = jnp.ones((CHUNK, HIDDEN), _f32)
    zeros_s = jnp.zeros((RPT, HIDDEN), _f32)
    I8 = jnp.eye(8, dtype=_f32)
    R0 = jnp.kron(jnp.eye(D_IN, dtype=_f32), jnp.ones((1, HIDDEN), _f32))
    S0 = jnp.kron(jnp.ones((D_IN, 1), _f32), jnp.eye(HIDDEN, dtype=_f32))
    R1 = jnp.kron(jnp.eye(HIDDEN, dtype=_f32), jnp.ones((1, HIDDEN), _f32))
    S1 = jnp.kron(jnp.ones((HIDDEN, 1), _f32), jnp.eye(HIDDEN, dtype=_f32))
    We0_8 = jnp.kron(I8, We0).astype(_bf16)
    We1_8 = jnp.kron(I8, We1).astype(_bf16)
    R0_8 = jnp.kron(I8, R0).astype(_bf16)
    R1_8 = jnp.kron(I8, R1).astype(_bf16)
    S0_8 = jnp.kron(I8, S0).astype(_bf16)
    S1_8 = jnp.kron(I8, S1).astype(_bf16)
    be0_8 = jnp.tile(be0, 8)
    be1_8 = jnp.tile(be1, 8)
    ea8 = edge_attr.reshape(E // 8, 128).astype(_bf16)
    pe = jnp.kron(jnp.eye(_B8, dtype=_bf16), jnp.array([[1, 0]], _bf16))
    po = jnp.kron(jnp.eye(_B8, dtype=_bf16), jnp.array([[0, 1]], _bf16))

    x8 = jnp.concatenate(
        [x, jnp.zeros((N_PAD - N, D_IN), _f32)]).reshape(_NP8, 8 * D_IN)
    root0_8 = jnp.kron(I8, root0)
    root1_8 = jnp.kron(I8, root1)
    bias0_8 = jnp.tile(bias0, 8)
    bias1_8 = jnp.tile(bias1, 8)
    btT = jnp.concatenate(
        [batch, jnp.full((N_PAD - N,), NUM_GRAPHS, jnp.int32)]
    ).reshape(_NP8, 8).T

    halves = (0, 1)
    off_b = E_HALF // _BE
    xgs = [_sc_gather_rows(x, srcg[hh], D_IN) for hh in halves]
    msg0 = [_edge_conv0(ea8, xgs[hh].reshape(E_HALF // 4, 128), pe, po,
                        We0_8, be0_8, R0_8, S0_8, hh * off_b)
            for hh in halves]
    sc0 = [_sc_scatter_count(msg0[hh].reshape(E_HALF, HIDDEN), dst3[hh],
                             ones_c, zeros_s) for hh in halves]
    s0_parts = [sc0[0][0], sc0[1][0]]
    cnt_parts = [sc0[0][1], sc0[1][1]]
    h8 = _node_update(s0_parts, cnt_parts, x8, root0_8, bias0_8)
    h_tab = h8.reshape(N_PAD, HIDDEN)
    hgs = [_sc_gather_rows(h_tab, srcg[hh], HIDDEN) for hh in halves]
    msg1 = [_edge_conv1(ea8, hgs[hh].reshape(E_HALF // 8, 128),
                        We1_8, be1_8, R1_8, S1_8, hh * off_b)
            for hh in halves]
    s1_parts = [_sc_scatter(msg1[hh].reshape(E_HALF, HIDDEN), dst3[hh],
                            zeros_s) for hh in halves]
    return _final(s1_parts, cnt_parts, h8, root1_8, bias1_8, btT,
                  fc0W, fc0b, fc1W, fc1b)
